# baseline (speedup 1.0000x reference)
import functools

import jax
import jax.numpy as jnp
from jax import lax
from jax.experimental import pallas as pl
from jax.experimental.pallas import tpu as pltpu
from jax.experimental.pallas import tpu_sc as plsc

BATCH = 16384
BAG = 20
TOTAL = BATCH * BAG
VOCAB = 100000
EMBEDDING_DIM = 3

_NC = 2
_NS = 16
_NW = _NC * _NS
_N_PER_W = TOTAL // _NW
_BAGS_PER_W = BATCH // _NW
_VP = 100352
_V_CHUNK = _VP // _NS
_EC = 16
_A_SUB = 1568
_B_CHUNK = 1280
_NB_CHUNK = _N_PER_W // _B_CHUNK


def _sc_body(idx_hbm, w_hbm, lshT_hbm, hw_hbm, out_hbm, emb_hbm,
             ci0, ci1, cv0, cv1, cv2, ibuf, ic0, ic1, w_v, val_a, val_b,
             outbuf, sem_a, sem_g, sem_w, sem_in, sem_b):
    cid = lax.axis_index("c")
    sid = lax.axis_index("s")
    wid = sid * _NC + cid
    lane = lax.iota(jnp.int32, 16)

    base = wid * _N_PER_W
    cp_w = pltpu.async_copy(w_hbm.at[pl.ds(base, _N_PER_W)], w_v, sem_w)

    v0 = sid * _V_CHUNK
    cis = (ci0, ci1)
    colvals = (cv0, cv1, cv2)
    cp_ci = pltpu.async_copy(lshT_hbm.at[0, pl.ds(v0, _V_CHUNK)], ci0, sem_a)
    for d in range(EMBEDDING_DIM):
        cp_ci.wait()
        if d + 1 < EMBEDDING_DIM:
            cp_ci = pltpu.async_copy(
                lshT_hbm.at[d + 1, pl.ds(v0, _V_CHUNK)], cis[(d + 1) % 2],
                sem_a)
        pltpu.async_copy(hw_hbm.at[cis[d % 2]], colvals[d], sem_g).wait()
    for c in range(_V_CHUNK // _A_SUB):
        for d in range(EMBEDDING_DIM):
            d_splat = jnp.full((16,), d, jnp.int32)

            def interleave(k, _, d=d, c=c, d_splat=d_splat):
                rows = k * 16 + lane
                vals = colvals[d][pl.ds(c * _A_SUB + k * 16, 16)]
                plsc.store_scatter(ibuf, [rows, d_splat], vals)
                return 0

            lax.fori_loop(0, _A_SUB // 16, interleave, 0)
        pltpu.sync_copy(ibuf, emb_hbm.at[pl.ds(v0 + c * _A_SUB, _A_SUB), :])
    plsc.subcore_barrier()

    ics = (ic0, ic1)
    bufs = (val_a, val_b)
    pltpu.async_copy(idx_hbm.at[pl.ds(base, _B_CHUNK)], ic0, sem_in).wait()
    g = pltpu.async_copy(emb_hbm.at[ic0], val_a, sem_b)
    cp_w.wait()
    for c in range(_NB_CHUNK):
        if c + 1 < _NB_CHUNK:
            cp_i = pltpu.async_copy(
                idx_hbm.at[pl.ds(base + (c + 1) * _B_CHUNK, _B_CHUNK)],
                ics[(c + 1) % 2], sem_in)
        g.wait()
        if c + 1 < _NB_CHUNK:
            cp_i.wait()
            g = pltpu.async_copy(emb_hbm.at[ics[(c + 1) % 2]],
                                 bufs[(c + 1) % 2], sem_b)
        cur = bufs[c % 2]

        def grp_step(gi, _, cur=cur, c=c):
            bags_local = gi * 16 + lane
            pos0 = bags_local * BAG
            for d in range(EMBEDDING_DIM):
                d_splat = jnp.full((16,), d, jnp.int32)
                acc = jnp.zeros((16,), jnp.float32)
                for j in range(BAG):
                    pos = pos0 + j
                    v = plsc.load_gather(cur, [pos, d_splat])
                    ww = plsc.load_gather(w_v, [c * _B_CHUNK + pos])
                    acc = acc + v * ww
                plsc.store_scatter(outbuf,
                                   [c * (_B_CHUNK // BAG) + bags_local,
                                    d_splat], acc)
            return 0

        lax.fori_loop(0, _B_CHUNK // BAG // 16, grp_step, 0)

    pltpu.sync_copy(outbuf, out_hbm.at[pl.ds(wid * _BAGS_PER_W, _BAGS_PER_W), :])


@jax.jit
def _lsh_embedding_bag(indices, per_index_weights, lshT, hashed_weight):
    mesh = plsc.VectorSubcoreMesh(core_axis_name="c", subcore_axis_name="s")
    grid_kernel = pl.kernel(
        _sc_body,
        out_type=(
            jax.ShapeDtypeStruct((BATCH, EMBEDDING_DIM), jnp.float32),
            jax.ShapeDtypeStruct((_VP, _EC), jnp.float32),
        ),
        mesh=mesh,
        compiler_params=pltpu.CompilerParams(
            use_tc_tiling_on_sc=False, needs_layout_passes=False),
        scratch_types=[
            pltpu.VMEM((_V_CHUNK,), jnp.int32),
            pltpu.VMEM((_V_CHUNK,), jnp.int32),
            pltpu.VMEM((_V_CHUNK,), jnp.float32),
            pltpu.VMEM((_V_CHUNK,), jnp.float32),
            pltpu.VMEM((_V_CHUNK,), jnp.float32),
            pltpu.VMEM((_A_SUB, _EC), jnp.float32),
            pltpu.VMEM((_B_CHUNK,), jnp.int32),
            pltpu.VMEM((_B_CHUNK,), jnp.int32),
            pltpu.VMEM((_N_PER_W,), jnp.float32),
            pltpu.VMEM((_B_CHUNK, _EC), jnp.float32),
            pltpu.VMEM((_B_CHUNK, _EC), jnp.float32),
            pltpu.VMEM((_BAGS_PER_W, EMBEDDING_DIM), jnp.float32),
            pltpu.SemaphoreType.DMA,
            pltpu.SemaphoreType.DMA,
            pltpu.SemaphoreType.DMA,
            pltpu.SemaphoreType.DMA,
            pltpu.SemaphoreType.DMA,
        ],
    )
    out, _ = grid_kernel(indices, per_index_weights, lshT, hashed_weight)
    return out


def kernel(indices, offsets, per_index_weights, hashed_weight,
           lsh_index_table):
    del offsets
    pad = jnp.zeros((_VP - VOCAB, EMBEDDING_DIM), jnp.int32)
    t = jnp.concatenate([lsh_index_table, pad], axis=0)
    lshT = t.T.copy()
    return _lsh_embedding_bag(indices, per_index_weights, lshT,
                              hashed_weight)


# speedup vs baseline: 1.4206x; 1.4206x over previous
import functools

import jax
import jax.numpy as jnp
from jax import lax
from jax.experimental import pallas as pl
from jax.experimental.pallas import tpu as pltpu
from jax.experimental.pallas import tpu_sc as plsc

BATCH = 16384
BAG = 20
TOTAL = BATCH * BAG
VOCAB = 100000
EMBEDDING_DIM = 3

_NC = 2
_NS = 16
_NW = _NC * _NS
_N_PER_W = TOTAL // _NW
_BAGS_PER_W = BATCH // _NW
_VP = 100352
_V_CHUNK = _VP // _NS


def _sc_body(idx_hbm, w_hbm, lshT_hbm, hw_hbm, out_hbm,
             ci0, ci1, cv0, cv1, cv2, pbuf, idx_v, w_v, val01, val2b,
             outbuf, p01, emb2, sem_in, sem_a, sem_g, sem_w, sem_b):
    cid = lax.axis_index("c")
    sid = lax.axis_index("s")
    wid = sid * _NC + cid
    lane = lax.iota(jnp.int32, 16)

    base = wid * _N_PER_W
    cp_idx = pltpu.async_copy(idx_hbm.at[pl.ds(base, _N_PER_W)], idx_v, sem_in)
    cp_w = pltpu.async_copy(w_hbm.at[pl.ds(base, _N_PER_W)], w_v, sem_in)

    v0 = sid * _V_CHUNK
    cis = (ci0, ci1)
    colvals = (cv0, cv1, cv2)
    cp_ci = pltpu.async_copy(lshT_hbm.at[0, pl.ds(v0, _V_CHUNK)], ci0, sem_a)
    for d in range(EMBEDDING_DIM):
        cp_ci.wait()
        if d + 1 < EMBEDDING_DIM:
            cp_ci = pltpu.async_copy(
                lshT_hbm.at[d + 1, pl.ds(v0, _V_CHUNK)], cis[(d + 1) % 2],
                sem_a)
        cp_g = pltpu.async_copy(hw_hbm.at[cis[d % 2]], colvals[d], sem_g)
        if d < 2:
            cp_g.wait()

    def pack_step(k, _):
        a = cv0[pl.ds(k * 16, 16)]
        b = cv1[pl.ds(k * 16, 16)]
        packed = plsc.pack(a, b, format=plsc.PackFormat.INTERLEAVED)
        pbuf[pl.ds(k * 16, 16)] = plsc.bitcast(packed, jnp.float32)
        return 0

    lax.fori_loop(0, _V_CHUNK // 16, pack_step, 0)
    cp_p = pltpu.async_copy(pbuf, p01.at[pl.ds(v0, _V_CHUNK)], sem_w)
    cp_g.wait()
    pltpu.async_copy(cv2, emb2.at[pl.ds(v0, _V_CHUNK)], sem_w).wait()
    cp_p.wait()
    plsc.subcore_barrier()

    cp_idx.wait()
    cp_w.wait()
    g01 = pltpu.async_copy(p01.at[idx_v], val01, sem_b)
    g2 = pltpu.async_copy(emb2.at[idx_v], val2b, sem_b)
    g01.wait()

    zero_splat = jnp.full((16,), 0, jnp.int32)
    one_splat = jnp.full((16,), 1, jnp.int32)
    two_splat = jnp.full((16,), 2, jnp.int32)

    def bag01_step(b16, _):
        bags = b16 * 16 + lane
        acc0 = jnp.zeros((16,), jnp.float32)
        acc1 = jnp.zeros((16,), jnp.float32)
        for j in range(BAG):
            pos = bags * BAG + j
            v01 = plsc.load_gather(val01, [pos])
            ww = plsc.load_gather(w_v, [pos])
            a, b = plsc.unpack(plsc.bitcast(v01, jnp.bfloat16),
                               format=plsc.PackFormat.INTERLEAVED)
            acc0 = acc0 + a * ww
            acc1 = acc1 + b * ww
        plsc.store_scatter(outbuf, [bags, zero_splat], acc0)
        plsc.store_scatter(outbuf, [bags, one_splat], acc1)
        return 0

    lax.fori_loop(0, _BAGS_PER_W // 16, bag01_step, 0)
    g2.wait()

    def bag2_step(b16, _):
        bags = b16 * 16 + lane
        acc = jnp.zeros((16,), jnp.float32)
        for j in range(BAG):
            pos = bags * BAG + j
            v = plsc.load_gather(val2b, [pos])
            ww = plsc.load_gather(w_v, [pos])
            acc = acc + v * ww
        plsc.store_scatter(outbuf, [bags, two_splat], acc)
        return 0

    lax.fori_loop(0, _BAGS_PER_W // 16, bag2_step, 0)

    pltpu.sync_copy(outbuf, out_hbm.at[pl.ds(wid * _BAGS_PER_W, _BAGS_PER_W), :])


@jax.jit
def _lsh_embedding_bag(indices, per_index_weights, lshT, hashed_weight):
    mesh = plsc.VectorSubcoreMesh(core_axis_name="c", subcore_axis_name="s")
    grid_kernel = pl.kernel(
        _sc_body,
        out_type=jax.ShapeDtypeStruct((BATCH, EMBEDDING_DIM), jnp.float32),
        mesh=mesh,
        compiler_params=pltpu.CompilerParams(
            use_tc_tiling_on_sc=False, needs_layout_passes=False),
        scratch_types=[
            pltpu.VMEM((_V_CHUNK,), jnp.int32),
            pltpu.VMEM((_V_CHUNK,), jnp.int32),
            pltpu.VMEM((_V_CHUNK,), jnp.float32),
            pltpu.VMEM((_V_CHUNK,), jnp.float32),
            pltpu.VMEM((_V_CHUNK,), jnp.float32),
            pltpu.VMEM((_V_CHUNK,), jnp.float32),
            pltpu.VMEM((_N_PER_W,), jnp.int32),
            pltpu.VMEM((_N_PER_W,), jnp.float32),
            pltpu.VMEM((_N_PER_W,), jnp.float32),
            pltpu.VMEM((_N_PER_W,), jnp.float32),
            pltpu.VMEM((_BAGS_PER_W, EMBEDDING_DIM), jnp.float32),
            pltpu.VMEM_SHARED((_VP,), jnp.float32),
            pltpu.VMEM_SHARED((_VP,), jnp.float32),
            pltpu.SemaphoreType.DMA,
            pltpu.SemaphoreType.DMA,
            pltpu.SemaphoreType.DMA,
            pltpu.SemaphoreType.DMA,
            pltpu.SemaphoreType.DMA,
        ],
    )
    return grid_kernel(indices, per_index_weights, lshT, hashed_weight)


def kernel(indices, offsets, per_index_weights, hashed_weight,
           lsh_index_table):
    del offsets
    pad = jnp.zeros((_VP - VOCAB, EMBEDDING_DIM), jnp.int32)
    t = jnp.concatenate([lsh_index_table, pad], axis=0)
    lshT = t.T.copy()
    return _lsh_embedding_bag(indices, per_index_weights, lshT,
                              hashed_weight)
